# Initial kernel scaffold; baseline (speedup 1.0000x reference)
import jax
import jax.numpy as jnp
from jax.experimental import pallas as pl


def kernel(rpn_bbox_deltas, rpn_labels, anchors):
    raise NotImplementedError("write your pallas kernel here")


# speedup vs baseline: 11.9197x; 11.9197x over previous
import functools

import jax
import jax.numpy as jnp
from jax.experimental import pallas as pl
from jax.experimental.pallas import tpu as pltpu

BATCH = 16
N = 20000
PRE = 6000
POST = 300
IOU_THR = 0.7
OUT_PAD = 384


def _monotone_key(scores):
    i = jax.lax.bitcast_convert_type(scores, jnp.int32)
    return jnp.where(i < 0, i ^ jnp.int32(0x7FFFFFFF), i)


def _nms_kernel(scores_ref, deltas_ref, anchors_ref, out_ref, s_ref, y1_ref,
                x1_ref, y2_ref, x2_ref, ab_ref):
    scores = scores_ref[...]
    key = _monotone_key(scores)
    lane = jax.lax.broadcasted_iota(jnp.int32, (BATCH, N), 1)

    def count_ge(thr):
        return jnp.sum((key >= thr).astype(jnp.int32), axis=1, keepdims=True)

    big = jnp.full((BATCH, 1), jnp.int32(-2147483648))
    zero = jnp.zeros((BATCH, 1), jnp.int32)
    cur = jnp.where(count_ge(zero) >= PRE, zero, big)

    def bis_body(k, cur):
        bit = jnp.int32(1) << (jnp.int32(30) - k)
        cand = cur | bit
        return jnp.where(count_ge(cand) >= PRE, cand, cur)

    thr = jax.lax.fori_loop(0, 31, bis_body, cur)

    gt = jnp.sum((key > thr).astype(jnp.int32), axis=1, keepdims=True)
    need = PRE - gt
    eq = key == thr

    def idx_body(k, cur):
        bit = jnp.int32(1) << (jnp.int32(14) - k)
        cand = cur | bit
        cnt = jnp.sum((eq & (lane < cand)).astype(jnp.int32), axis=1,
                      keepdims=True)
        return jnp.where(cnt < need, cand, cur)

    idx_thr = jax.lax.fori_loop(0, 15, idx_body,
                                jnp.zeros((BATCH, 1), jnp.int32))
    valid = (key > thr) | (eq & (lane <= idx_thr))

    a_y1 = anchors_ref[0:1, :]
    a_x1 = anchors_ref[1:2, :]
    a_y2 = anchors_ref[2:3, :]
    a_x2 = anchors_ref[3:4, :]
    anc_w = a_x2 - a_x1
    anc_h = a_y2 - a_y1
    anc_cx = a_x1 + 0.5 * anc_w
    anc_cy = a_y1 + 0.5 * anc_h
    d_y = deltas_ref[0] * jnp.float32(0.1)
    d_x = deltas_ref[1] * jnp.float32(0.1)
    d_h = deltas_ref[2] * jnp.float32(0.2)
    d_w = deltas_ref[3] * jnp.float32(0.2)
    bb_w = jnp.exp(d_w) * anc_w
    bb_h = jnp.exp(d_h) * anc_h
    bb_cx = d_x * anc_w + anc_cx
    bb_cy = d_y * anc_h + anc_cy
    y1 = bb_cy - 0.5 * bb_h
    x1 = bb_cx - 0.5 * bb_w
    y2 = bb_h + y1
    x2 = bb_w + x1

    s_ref[...] = jnp.where(valid, scores, jnp.float32(-1.0))
    y1_ref[...] = y1
    x1_ref[...] = x1
    y2_ref[...] = y2
    x2_ref[...] = x2
    ab_ref[...] = jnp.maximum(y2 - y1, 0.0) * jnp.maximum(x2 - x1, 0.0)
    out_ref[...] = jnp.zeros((4, BATCH, OUT_PAD), jnp.float32)

    out_lane = jax.lax.broadcasted_iota(jnp.int32, (BATCH, OUT_PAD), 1)

    def body(i, _):
        s = s_ref[...]
        m = jnp.max(s, axis=1, keepdims=True)
        anyv = m >= 0.0
        pick = (s == m) & anyv
        pos = jnp.min(jnp.where(pick, lane, jnp.int32(N)), axis=1,
                      keepdims=True)
        onehot = lane == pos

        def sel(plane):
            return jnp.sum(jnp.where(onehot, plane, 0.0), axis=1,
                           keepdims=True)

        by1 = sel(y1_ref[...])
        bx1 = sel(x1_ref[...])
        by2 = sel(y2_ref[...])
        bx2 = sel(x2_ref[...])

        yy1 = jnp.maximum(by1, y1_ref[...])
        xx1 = jnp.maximum(bx1, x1_ref[...])
        yy2 = jnp.minimum(by2, y2_ref[...])
        xx2 = jnp.minimum(bx2, x2_ref[...])
        inter = jnp.maximum(yy2 - yy1, 0.0) * jnp.maximum(xx2 - xx1, 0.0)
        area_a = jnp.maximum(by2 - by1, 0.0) * jnp.maximum(bx2 - bx1, 0.0)
        iou = inter / jnp.maximum(area_a + ab_ref[...] - inter, 1e-8)
        supp = (iou > IOU_THR) | onehot
        s_ref[...] = jnp.where(anyv & supp, jnp.float32(-1.0), s)

        wmask = (out_lane == i) & anyv
        o = out_ref[...]
        o0 = jnp.where(wmask, jnp.clip(by1, 0.0, 1.0), o[0])
        o1 = jnp.where(wmask, jnp.clip(bx1, 0.0, 1.0), o[1])
        o2 = jnp.where(wmask, jnp.clip(by2, 0.0, 1.0), o[2])
        o3 = jnp.where(wmask, jnp.clip(bx2, 0.0, 1.0), o[3])
        out_ref[...] = jnp.stack([o0, o1, o2, o3], axis=0)
        return 0

    jax.lax.fori_loop(0, POST, body, 0)


@jax.jit
def kernel(rpn_bbox_deltas, rpn_labels, anchors):
    deltas_t = jnp.transpose(rpn_bbox_deltas, (2, 0, 1))
    anchors_t = jnp.transpose(anchors, (1, 0))
    out = pl.pallas_call(
        _nms_kernel,
        out_shape=jax.ShapeDtypeStruct((4, BATCH, OUT_PAD), jnp.float32),
        scratch_shapes=[pltpu.VMEM((BATCH, N), jnp.float32)] * 6,
    )(rpn_labels, deltas_t, anchors_t)
    return jnp.transpose(out[:, :, :POST], (1, 2, 0))


# speedup vs baseline: 15.3789x; 1.2902x over previous
import functools

import jax
import jax.numpy as jnp
from jax.experimental import pallas as pl
from jax.experimental.pallas import tpu as pltpu

BATCH = 16
N = 20000
PRE = 6000
POST = 300
IOU_THR = 0.7
OUT_PAD = 384


def _monotone_key(scores):
    i = jax.lax.bitcast_convert_type(scores, jnp.int32)
    return jnp.where(i < 0, i ^ jnp.int32(0x7FFFFFFF), i)


def _nms_kernel(scores_ref, deltas_ref, anchors_ref, out_ref, s_ref, y1_ref,
                x1_ref, y2_ref, x2_ref, ab_ref):
    scores = scores_ref[...]
    key = _monotone_key(scores)
    lane = jax.lax.broadcasted_iota(jnp.int32, (BATCH, N), 1)

    def count_ge(thr):
        return jnp.sum((key >= thr).astype(jnp.int32), axis=1, keepdims=True)

    big = jnp.full((BATCH, 1), jnp.int32(-2147483648))
    zero = jnp.zeros((BATCH, 1), jnp.int32)
    cur = jnp.where(count_ge(zero) >= PRE, zero, big)

    def bis_body(k, cur):
        bit = jnp.int32(1) << (jnp.int32(30) - k)
        cand = cur | bit
        return jnp.where(count_ge(cand) >= PRE, cand, cur)

    thr = jax.lax.fori_loop(0, 31, bis_body, cur)

    gt = jnp.sum((key > thr).astype(jnp.int32), axis=1, keepdims=True)
    need = PRE - gt
    eq = key == thr

    def idx_body(k, cur):
        bit = jnp.int32(1) << (jnp.int32(14) - k)
        cand = cur | bit
        cnt = jnp.sum((eq & (lane < cand)).astype(jnp.int32), axis=1,
                      keepdims=True)
        return jnp.where(cnt < need, cand, cur)

    idx_thr = jax.lax.fori_loop(0, 15, idx_body,
                                jnp.zeros((BATCH, 1), jnp.int32))
    valid = (key > thr) | (eq & (lane <= idx_thr))

    a_y1 = anchors_ref[0:1, :]
    a_x1 = anchors_ref[1:2, :]
    a_y2 = anchors_ref[2:3, :]
    a_x2 = anchors_ref[3:4, :]
    anc_w = a_x2 - a_x1
    anc_h = a_y2 - a_y1
    anc_cx = a_x1 + 0.5 * anc_w
    anc_cy = a_y1 + 0.5 * anc_h
    d_y = deltas_ref[0] * jnp.float32(0.1)
    d_x = deltas_ref[1] * jnp.float32(0.1)
    d_h = deltas_ref[2] * jnp.float32(0.2)
    d_w = deltas_ref[3] * jnp.float32(0.2)
    bb_w = jnp.exp(d_w) * anc_w
    bb_h = jnp.exp(d_h) * anc_h
    bb_cx = d_x * anc_w + anc_cx
    bb_cy = d_y * anc_h + anc_cy
    y1 = bb_cy - 0.5 * bb_h
    x1 = bb_cx - 0.5 * bb_w
    y2 = bb_h + y1
    x2 = bb_w + x1

    s_ref[...] = jnp.where(valid, scores, jnp.float32(-1.0))
    y1_ref[...] = y1
    x1_ref[...] = x1
    y2_ref[...] = y2
    x2_ref[...] = x2
    ab_ref[...] = jnp.maximum(y2 - y1, 0.0) * jnp.maximum(x2 - x1, 0.0)
    out_ref[...] = jnp.zeros((4, BATCH, OUT_PAD), jnp.float32)

    out_lane = jax.lax.broadcasted_iota(jnp.int32, (BATCH, OUT_PAD), 1)
    yy1 = y1_ref[...]
    xx1 = x1_ref[...]
    yy2 = y2_ref[...]
    xx2 = x2_ref[...]
    ab = ab_ref[...]

    def one_pick(s):
        m = jnp.max(s, axis=1, keepdims=True)
        anyv = m >= 0.0
        pick = (s == m) & anyv
        pos = jnp.min(jnp.where(pick, lane, jnp.int32(N)), axis=1,
                      keepdims=True)
        onehot = lane == pos

        def sel(plane):
            return jnp.sum(jnp.where(onehot, plane, 0.0), axis=1,
                           keepdims=True)

        by1 = sel(yy1)
        bx1 = sel(xx1)
        by2 = sel(yy2)
        bx2 = sel(xx2)

        inter = (jnp.maximum(jnp.minimum(by2, yy2) - jnp.maximum(by1, yy1),
                             0.0)
                 * jnp.maximum(jnp.minimum(bx2, xx2) - jnp.maximum(bx1, xx1),
                               0.0))
        area_a = jnp.maximum(by2 - by1, 0.0) * jnp.maximum(bx2 - bx1, 0.0)
        supp = inter > IOU_THR * jnp.maximum(area_a + ab - inter, 1e-8)
        supp = supp | onehot
        s_next = jnp.where(anyv & supp, jnp.float32(-1.0), s)
        return s_next, (by1, bx1, by2, bx2), anyv

    def write_out(o, i, box, anyv):
        wmask = (out_lane == i) & anyv
        return [jnp.where(wmask, jnp.clip(b, 0.0, 1.0), oo)
                for b, oo in zip(box, o)]

    def body(i, _):
        s = s_ref[...]
        s, box1, anyv1 = one_pick(s)
        s, box2, anyv2 = one_pick(s)
        s_ref[...] = s
        o = out_ref[...]
        o = write_out([o[0], o[1], o[2], o[3]], 2 * i, box1, anyv1)
        o = write_out(o, 2 * i + 1, box2, anyv2)
        out_ref[...] = jnp.stack(o, axis=0)
        return 0

    jax.lax.fori_loop(0, POST // 2, body, 0)


@jax.jit
def kernel(rpn_bbox_deltas, rpn_labels, anchors):
    deltas_t = jnp.transpose(rpn_bbox_deltas, (2, 0, 1))
    anchors_t = jnp.transpose(anchors, (1, 0))
    out = pl.pallas_call(
        _nms_kernel,
        out_shape=jax.ShapeDtypeStruct((4, BATCH, OUT_PAD), jnp.float32),
        scratch_shapes=[pltpu.VMEM((BATCH, N), jnp.float32)] * 6,
    )(rpn_labels, deltas_t, anchors_t)
    return jnp.transpose(out[:, :, :POST], (1, 2, 0))


# speedup vs baseline: 18.2639x; 1.1876x over previous
import functools

import jax
import jax.numpy as jnp
from jax.experimental import pallas as pl
from jax.experimental.pallas import tpu as pltpu

BATCH = 16
N = 20000
PRE = 6000
POST = 300
IOU_THR = 0.7
OUT_PAD = 384


def _monotone_key(scores):
    i = jax.lax.bitcast_convert_type(scores, jnp.int32)
    return jnp.where(i < 0, i ^ jnp.int32(0x7FFFFFFF), i)


def _nms_kernel(scores_ref, deltas_ref, anchors_ref, out_ref, s_ref, y1_ref,
                x1_ref, y2_ref, x2_ref, ab_ref):
    scores = scores_ref[...]
    key = _monotone_key(scores)
    lane = jax.lax.broadcasted_iota(jnp.int32, (BATCH, N), 1)

    def count_ge(thr):
        return jnp.sum((key >= thr).astype(jnp.int32), axis=1, keepdims=True)

    big = jnp.full((BATCH, 1), jnp.int32(-2147483648))
    zero = jnp.zeros((BATCH, 1), jnp.int32)
    cur = jnp.where(count_ge(zero) >= PRE, zero, big)

    def bis_body(k, cur):
        bit = jnp.int32(1) << (jnp.int32(30) - k)
        cand = cur | bit
        return jnp.where(count_ge(cand) >= PRE, cand, cur)

    thr = jax.lax.fori_loop(0, 31, bis_body, cur)

    gt = jnp.sum((key > thr).astype(jnp.int32), axis=1, keepdims=True)
    need = PRE - gt
    eq = key == thr

    def idx_body(k, cur):
        bit = jnp.int32(1) << (jnp.int32(14) - k)
        cand = cur | bit
        cnt = jnp.sum((eq & (lane < cand)).astype(jnp.int32), axis=1,
                      keepdims=True)
        return jnp.where(cnt < need, cand, cur)

    idx_thr = jax.lax.fori_loop(0, 15, idx_body,
                                jnp.zeros((BATCH, 1), jnp.int32))
    valid = (key > thr) | (eq & (lane <= idx_thr))

    a_y1 = anchors_ref[0:1, :]
    a_x1 = anchors_ref[1:2, :]
    a_y2 = anchors_ref[2:3, :]
    a_x2 = anchors_ref[3:4, :]
    anc_w = a_x2 - a_x1
    anc_h = a_y2 - a_y1
    anc_cx = a_x1 + 0.5 * anc_w
    anc_cy = a_y1 + 0.5 * anc_h
    d_y = deltas_ref[0] * jnp.float32(0.1)
    d_x = deltas_ref[1] * jnp.float32(0.1)
    d_h = deltas_ref[2] * jnp.float32(0.2)
    d_w = deltas_ref[3] * jnp.float32(0.2)
    bb_w = jnp.exp(d_w) * anc_w
    bb_h = jnp.exp(d_h) * anc_h
    bb_cx = d_x * anc_w + anc_cx
    bb_cy = d_y * anc_h + anc_cy
    y1 = bb_cy - 0.5 * bb_h
    x1 = bb_cx - 0.5 * bb_w
    y2 = bb_h + y1
    x2 = bb_w + x1

    s_ref[...] = jnp.where(valid, scores, jnp.float32(-1.0))
    y1_ref[...] = y1
    x1_ref[...] = x1
    y2_ref[...] = y2
    x2_ref[...] = x2
    ab_ref[...] = (jnp.float32(IOU_THR)
                   * jnp.maximum(y2 - y1, 0.0) * jnp.maximum(x2 - x1, 0.0))
    out_ref[...] = jnp.zeros((4, BATCH, OUT_PAD), jnp.float32)

    out_lane = jax.lax.broadcasted_iota(jnp.int32, (BATCH, OUT_PAD), 1)
    yy1 = y1_ref[...]
    xx1 = x1_ref[...]
    yy2 = y2_ref[...]
    xx2 = x2_ref[...]
    ab = ab_ref[...]

    def one_pick(s):
        m = jnp.max(s, axis=1, keepdims=True)
        anyv = m >= 0.0
        pick = (s == m) & anyv
        pos = jnp.min(jnp.where(pick, lane, jnp.int32(N)), axis=1,
                      keepdims=True)
        onehot = lane == pos

        def sel(plane):
            return jnp.sum(jnp.where(onehot, plane, 0.0), axis=1,
                           keepdims=True)

        by1 = sel(yy1)
        bx1 = sel(xx1)
        by2 = sel(yy2)
        bx2 = sel(xx2)

        inter = (jnp.maximum(jnp.minimum(by2, yy2) - jnp.maximum(by1, yy1),
                             0.0)
                 * jnp.maximum(jnp.minimum(bx2, xx2) - jnp.maximum(bx1, xx1),
                               0.0))
        area_a = jnp.maximum(by2 - by1, 0.0) * jnp.maximum(bx2 - bx1, 0.0)
        supp = (jnp.float32(1.0 + IOU_THR) * inter
                > jnp.float32(IOU_THR) * area_a + ab)
        supp = supp | onehot
        s_next = jnp.where(anyv & supp, jnp.float32(-1.0), s)
        return s_next, (by1, bx1, by2, bx2), anyv

    def write_out(o, i, box, anyv):
        wmask = (out_lane == i) & anyv
        return [jnp.where(wmask, jnp.clip(b, 0.0, 1.0), oo)
                for b, oo in zip(box, o)]

    def body(i, _):
        s = s_ref[...]
        o = [out_ref[0], out_ref[1], out_ref[2], out_ref[3]]
        for k in range(4):
            s, box, anyv = one_pick(s)
            o = write_out(o, 4 * i + k, box, anyv)
        s_ref[...] = s
        out_ref[...] = jnp.stack(o, axis=0)
        return 0

    jax.lax.fori_loop(0, POST // 4, body, 0)


@jax.jit
def kernel(rpn_bbox_deltas, rpn_labels, anchors):
    deltas_t = jnp.transpose(rpn_bbox_deltas, (2, 0, 1))
    anchors_t = jnp.transpose(anchors, (1, 0))
    out = pl.pallas_call(
        _nms_kernel,
        out_shape=jax.ShapeDtypeStruct((4, BATCH, OUT_PAD), jnp.float32),
        scratch_shapes=[pltpu.VMEM((BATCH, N), jnp.float32)] * 6,
    )(rpn_labels, deltas_t, anchors_t)
    return jnp.transpose(out[:, :, :POST], (1, 2, 0))


# speedup vs baseline: 18.4415x; 1.0097x over previous
import functools

import jax
import jax.numpy as jnp
from jax.experimental import pallas as pl
from jax.experimental.pallas import tpu as pltpu

BATCH = 16
N = 20000
PRE = 6000
POST = 300
IOU_THR = 0.7
OUT_PAD = 384


def _monotone_key(scores):
    i = jax.lax.bitcast_convert_type(scores, jnp.int32)
    return jnp.where(i < 0, i ^ jnp.int32(0x7FFFFFFF), i)


def _nms_kernel(scores_ref, deltas_ref, anchors_ref, out_ref, s_ref, y1_ref,
                x1_ref, y2_ref, x2_ref, ab_ref):
    scores = scores_ref[...]
    key = _monotone_key(scores)
    lane = jax.lax.broadcasted_iota(jnp.int32, (BATCH, N), 1)

    def count_ge(thr):
        return jnp.sum((key >= thr).astype(jnp.int32), axis=1, keepdims=True)

    big = jnp.full((BATCH, 1), jnp.int32(-2147483648))
    zero = jnp.zeros((BATCH, 1), jnp.int32)
    cur = jnp.where(count_ge(zero) >= PRE, zero, big)

    def bis_body(k, cur):
        bit = jnp.int32(1) << (jnp.int32(30) - k)
        cand = cur | bit
        return jnp.where(count_ge(cand) >= PRE, cand, cur)

    thr = jax.lax.fori_loop(0, 31, bis_body, cur)

    gt = jnp.sum((key > thr).astype(jnp.int32), axis=1, keepdims=True)
    need = PRE - gt
    eq = key == thr

    def idx_body(k, cur):
        bit = jnp.int32(1) << (jnp.int32(14) - k)
        cand = cur | bit
        cnt = jnp.sum((eq & (lane < cand)).astype(jnp.int32), axis=1,
                      keepdims=True)
        return jnp.where(cnt < need, cand, cur)

    idx_thr = jax.lax.fori_loop(0, 15, idx_body,
                                jnp.zeros((BATCH, 1), jnp.int32))
    valid = (key > thr) | (eq & (lane <= idx_thr))

    a_y1 = anchors_ref[0:1, :]
    a_x1 = anchors_ref[1:2, :]
    a_y2 = anchors_ref[2:3, :]
    a_x2 = anchors_ref[3:4, :]
    anc_w = a_x2 - a_x1
    anc_h = a_y2 - a_y1
    anc_cx = a_x1 + 0.5 * anc_w
    anc_cy = a_y1 + 0.5 * anc_h
    d_y = deltas_ref[0] * jnp.float32(0.1)
    d_x = deltas_ref[1] * jnp.float32(0.1)
    d_h = deltas_ref[2] * jnp.float32(0.2)
    d_w = deltas_ref[3] * jnp.float32(0.2)
    bb_w = jnp.exp(d_w) * anc_w
    bb_h = jnp.exp(d_h) * anc_h
    bb_cx = d_x * anc_w + anc_cx
    bb_cy = d_y * anc_h + anc_cy
    y1 = bb_cy - 0.5 * bb_h
    x1 = bb_cx - 0.5 * bb_w
    y2 = bb_h + y1
    x2 = bb_w + x1

    zf = jnp.float32(0.0)
    s_ref[...] = jnp.where(valid, scores, jnp.float32(-1.0))
    y1_ref[...] = jnp.where(valid, y1, zf)
    x1_ref[...] = jnp.where(valid, x1, zf)
    y2_ref[...] = jnp.where(valid, y2, zf)
    x2_ref[...] = jnp.where(valid, x2, zf)
    ab_ref[...] = jnp.where(
        valid,
        jnp.float32(IOU_THR)
        * jnp.maximum(y2 - y1, 0.0) * jnp.maximum(x2 - x1, 0.0), zf)
    out_ref[...] = jnp.zeros((4, BATCH, OUT_PAD), jnp.float32)

    out_lane = jax.lax.broadcasted_iota(jnp.int32, (BATCH, OUT_PAD), 1)
    yy1 = y1_ref[...]
    xx1 = x1_ref[...]
    yy2 = y2_ref[...]
    xx2 = x2_ref[...]
    ab = ab_ref[...]

    def one_pick(s):
        m = jnp.max(s, axis=1, keepdims=True)
        anyv = m >= 0.0
        pick = (s == m) & anyv
        pos = jnp.min(jnp.where(pick, lane, jnp.int32(N)), axis=1,
                      keepdims=True)
        onehot = lane == pos
        oh_f = jnp.where(onehot, jnp.float32(1.0), jnp.float32(0.0))

        def sel(plane):
            return jnp.sum(plane * oh_f, axis=1, keepdims=True)

        by1 = sel(yy1)
        bx1 = sel(xx1)
        by2 = sel(yy2)
        bx2 = sel(xx2)

        inter = (jnp.maximum(jnp.minimum(by2, yy2) - jnp.maximum(by1, yy1),
                             0.0)
                 * jnp.maximum(jnp.minimum(bx2, xx2) - jnp.maximum(bx1, xx1),
                               0.0))
        area_a = jnp.maximum(by2 - by1, 0.0) * jnp.maximum(bx2 - bx1, 0.0)
        supp = (jnp.float32(1.0 + IOU_THR) * inter
                > jnp.float32(IOU_THR) * area_a + ab)
        supp = supp | onehot
        s_next = jnp.where(anyv & supp, jnp.float32(-1.0), s)
        return s_next, (by1, bx1, by2, bx2), anyv

    def write_out(o, i, box, anyv):
        wmask = (out_lane == i) & anyv
        return [jnp.where(wmask, jnp.clip(b, 0.0, 1.0), oo)
                for b, oo in zip(box, o)]

    def body(i, _):
        s = s_ref[...]
        o = [out_ref[0], out_ref[1], out_ref[2], out_ref[3]]
        for k in range(4):
            s, box, anyv = one_pick(s)
            o = write_out(o, 4 * i + k, box, anyv)
        s_ref[...] = s
        out_ref[...] = jnp.stack(o, axis=0)
        return 0

    jax.lax.fori_loop(0, POST // 4, body, 0)


@jax.jit
def kernel(rpn_bbox_deltas, rpn_labels, anchors):
    deltas_t = jnp.transpose(rpn_bbox_deltas, (2, 0, 1))
    anchors_t = jnp.transpose(anchors, (1, 0))
    out = pl.pallas_call(
        _nms_kernel,
        out_shape=jax.ShapeDtypeStruct((4, BATCH, OUT_PAD), jnp.float32),
        scratch_shapes=[pltpu.VMEM((BATCH, N), jnp.float32)] * 6,
    )(rpn_labels, deltas_t, anchors_t)
    return jnp.transpose(out[:, :, :POST], (1, 2, 0))


# speedup vs baseline: 40.3681x; 2.1890x over previous
import functools

import jax
import jax.numpy as jnp
from jax.experimental import pallas as pl
from jax.experimental.pallas import tpu as pltpu

BATCH = 16
N = 20000
PRE = 6000
POST = 300
IOU_THR = 0.7
OUT_PAD = 384
PADN = 20480
BLKS = 160
OROWS = 48
CW = OROWS * 128


def _monotone_key(scores):
    i = jax.lax.bitcast_convert_type(scores, jnp.int32)
    return jnp.where(i < 0, i ^ jnp.int32(0x7FFFFFFF), i)


def _nms_kernel(scores_ref, deltas_ref, anchors_ref, out_ref, s_ref):
    scores = scores_ref[...]
    key = _monotone_key(scores)
    lane = jax.lax.broadcasted_iota(jnp.int32, (BATCH, N), 1)

    def count_ge(thr):
        return jnp.sum((key >= thr).astype(jnp.int32), axis=1, keepdims=True)

    big = jnp.full((BATCH, 1), jnp.int32(-2147483648))
    zero = jnp.zeros((BATCH, 1), jnp.int32)
    cur = jnp.where(count_ge(zero) >= PRE, zero, big)

    def bis_body(k, cur):
        bit = jnp.int32(1) << (jnp.int32(30) - k)
        cand = cur | bit
        return jnp.where(count_ge(cand) >= PRE, cand, cur)

    thr = jax.lax.fori_loop(0, 31, bis_body, cur)

    gt = jnp.sum((key > thr).astype(jnp.int32), axis=1, keepdims=True)
    need = PRE - gt
    eq = key == thr

    def idx_body(k, cur):
        bit = jnp.int32(1) << (jnp.int32(14) - k)
        cand = cur | bit
        cnt = jnp.sum((eq & (lane < cand)).astype(jnp.int32), axis=1,
                      keepdims=True)
        return jnp.where(cnt < need, cand, cur)

    idx_thr = jax.lax.fori_loop(0, 15, idx_body,
                                jnp.zeros((BATCH, 1), jnp.int32))
    valid = (key > thr) | (eq & (lane <= idx_thr))

    a_y1 = anchors_ref[0:1, :]
    a_x1 = anchors_ref[1:2, :]
    a_y2 = anchors_ref[2:3, :]
    a_x2 = anchors_ref[3:4, :]
    anc_w = a_x2 - a_x1
    anc_h = a_y2 - a_y1
    anc_cx = a_x1 + 0.5 * anc_w
    anc_cy = a_y1 + 0.5 * anc_h
    d_y = deltas_ref[0] * jnp.float32(0.1)
    d_x = deltas_ref[1] * jnp.float32(0.1)
    d_h = deltas_ref[2] * jnp.float32(0.2)
    d_w = deltas_ref[3] * jnp.float32(0.2)
    bb_w = jnp.exp(d_w) * anc_w
    bb_h = jnp.exp(d_h) * anc_h
    bb_cx = d_x * anc_w + anc_cx
    bb_cy = d_y * anc_h + anc_cy
    y1 = bb_cy - 0.5 * bb_h
    x1 = bb_cx - 0.5 * bb_w
    y2 = bb_h + y1
    x2 = bb_w + x1

    zf = jnp.float32(0.0)
    pad = jnp.zeros((BATCH, PADN - N), jnp.float32)
    vf = jnp.concatenate([jnp.where(valid, jnp.float32(1.0), zf), pad],
                         axis=1)
    V = vf.reshape(BATCH * BLKS, 128)
    li = jax.lax.broadcasted_iota(jnp.int32, (128, 128), 0)
    lj = jax.lax.broadcasted_iota(jnp.int32, (128, 128), 1)
    T128 = jnp.where(li < lj, jnp.float32(1.0), zf)
    rank_ex = jax.lax.dot_general(
        V, T128, (((1,), (0,)), ((), ())),
        preferred_element_type=jnp.float32,
        precision=jax.lax.Precision.HIGHEST)
    kidx_f = jax.lax.broadcasted_iota(
        jnp.int32, (BATCH * BLKS, 128), 1).astype(jnp.float32)
    lo = jnp.zeros((BATCH * BLKS, 128), jnp.int32)
    for bit in (64, 32, 16, 8, 4, 2, 1):
        c = lo + bit
        rc = jnp.take_along_axis(rank_ex, c, axis=1)
        lo = jnp.where(rc <= kidx_f, c, lo)
    cnt = rank_ex[:, 127:128] + V[:, 127:128]
    keep = kidx_f < cnt

    C = jnp.sum(vf.reshape(BATCH, BLKS, 128), axis=2)
    i160 = jax.lax.broadcasted_iota(jnp.int32, (BLKS, BLKS), 0)
    j160 = jax.lax.broadcasted_iota(jnp.int32, (BLKS, BLKS), 1)
    T160 = jnp.where(i160 < j160, jnp.float32(1.0), zf)
    O = jax.lax.dot_general(
        C, T160, (((1,), (0,)), ((), ())),
        preferred_element_type=jnp.float32,
        precision=jax.lax.Precision.HIGHEST)
    Oi = O.astype(jnp.int32)
    shift = jnp.broadcast_to((Oi & 127)[:, :, None],
                             (BATCH, BLKS, 128)).reshape(BATCH * BLKS, 128)
    lane128 = jax.lax.broadcasted_iota(jnp.int32, (BATCH * BLKS, 128), 1)
    idxrot = (lane128 - shift + 128) & 127
    real = idxrot.astype(jnp.float32) < cnt
    partA = real & (lane128 >= shift)
    partB = real & (lane128 < shift)
    m0 = Oi >> 7
    mm = jax.lax.broadcasted_iota(jnp.int32, (BATCH, BLKS, OROWS), 2)
    RA = jnp.where(m0[:, :, None] == mm, jnp.float32(1.0), zf)
    RB = jnp.where((m0 + 1)[:, :, None] == mm, jnp.float32(1.0), zf)

    def compact(p):
        pp = jnp.concatenate([p, pad], axis=1).reshape(BATCH * BLKS, 128)
        ph1 = jnp.where(keep, jnp.take_along_axis(pp, lo, axis=1), zf)
        rot = jnp.take_along_axis(ph1, idxrot, axis=1)
        mA = jnp.where(partA, rot, zf).reshape(BATCH, BLKS, 128)
        mB = jnp.where(partB, rot, zf).reshape(BATCH, BLKS, 128)
        o3 = (jax.lax.dot_general(
                  RA, mA, (((1,), (1,)), ((0,), (0,))),
                  preferred_element_type=jnp.float32,
                  precision=jax.lax.Precision.HIGHEST)
              + jax.lax.dot_general(
                  RB, mB, (((1,), (1,)), ((0,), (0,))),
                  preferred_element_type=jnp.float32,
                  precision=jax.lax.Precision.HIGHEST))
        return o3.reshape(BATCH, CW)

    clane = jax.lax.broadcasted_iota(jnp.int32, (BATCH, CW), 1)
    yy1 = compact(y1)
    xx1 = compact(x1)
    yy2 = compact(y2)
    xx2 = compact(x2)
    s_c = jnp.where(clane < PRE, compact(scores), jnp.float32(-1.0))
    ab = (jnp.float32(IOU_THR)
          * jnp.maximum(yy2 - yy1, 0.0) * jnp.maximum(xx2 - xx1, 0.0))

    s_ref[...] = s_c
    out_ref[...] = jnp.zeros((4, BATCH, OUT_PAD), jnp.float32)
    out_lane = jax.lax.broadcasted_iota(jnp.int32, (BATCH, OUT_PAD), 1)
    lane = clane

    def one_pick(s):
        m = jnp.max(s, axis=1, keepdims=True)
        anyv = m >= 0.0
        pick = (s == m) & anyv
        pos = jnp.min(jnp.where(pick, lane, jnp.int32(CW)), axis=1,
                      keepdims=True)
        onehot = lane == pos
        oh_f = jnp.where(onehot, jnp.float32(1.0), jnp.float32(0.0))

        def sel(plane):
            return jnp.sum(plane * oh_f, axis=1, keepdims=True)

        by1 = sel(yy1)
        bx1 = sel(xx1)
        by2 = sel(yy2)
        bx2 = sel(xx2)

        inter = (jnp.maximum(jnp.minimum(by2, yy2) - jnp.maximum(by1, yy1),
                             0.0)
                 * jnp.maximum(jnp.minimum(bx2, xx2) - jnp.maximum(bx1, xx1),
                               0.0))
        area_a = jnp.maximum(by2 - by1, 0.0) * jnp.maximum(bx2 - bx1, 0.0)
        supp = (jnp.float32(1.0 + IOU_THR) * inter
                > jnp.float32(IOU_THR) * area_a + ab)
        supp = supp | onehot
        s_next = jnp.where(anyv & supp, jnp.float32(-1.0), s)
        return s_next, (by1, bx1, by2, bx2), anyv

    def write_out(o, i, box, anyv):
        wmask = (out_lane == i) & anyv
        return [jnp.where(wmask, jnp.clip(b, 0.0, 1.0), oo)
                for b, oo in zip(box, o)]

    def body(i, _):
        s = s_ref[...]
        o = [out_ref[0], out_ref[1], out_ref[2], out_ref[3]]
        for k in range(4):
            s, box, anyv = one_pick(s)
            o = write_out(o, 4 * i + k, box, anyv)
        s_ref[...] = s
        out_ref[...] = jnp.stack(o, axis=0)
        return 0

    jax.lax.fori_loop(0, POST // 4, body, 0)


@jax.jit
def kernel(rpn_bbox_deltas, rpn_labels, anchors):
    deltas_t = jnp.transpose(rpn_bbox_deltas, (2, 0, 1))
    anchors_t = jnp.transpose(anchors, (1, 0))
    out = pl.pallas_call(
        _nms_kernel,
        out_shape=jax.ShapeDtypeStruct((4, BATCH, OUT_PAD), jnp.float32),
        scratch_shapes=[pltpu.VMEM((BATCH, CW), jnp.float32)],
    )(rpn_labels, deltas_t, anchors_t)
    return jnp.transpose(out[:, :, :POST], (1, 2, 0))


# speedup vs baseline: 41.1456x; 1.0193x over previous
import functools

import jax
import jax.numpy as jnp
from jax.experimental import pallas as pl
from jax.experimental.pallas import tpu as pltpu

BATCH = 16
N = 20000
PRE = 6000
POST = 300
IOU_THR = 0.7
OUT_PAD = 384
PADN = 20480
BLKS = 160
OROWS = 48
CW = OROWS * 128


def _monotone_key(scores):
    i = jax.lax.bitcast_convert_type(scores, jnp.int32)
    return jnp.where(i < 0, i ^ jnp.int32(0x7FFFFFFF), i)


def _nms_kernel(scores_ref, deltas_ref, anchors_ref, out_ref, s_ref):
    scores = scores_ref[...]
    key = _monotone_key(scores)
    lane = jax.lax.broadcasted_iota(jnp.int32, (BATCH, N), 1)

    def count_ge(thr):
        return jnp.sum((key >= thr).astype(jnp.int32), axis=1, keepdims=True)

    big = jnp.full((BATCH, 1), jnp.int32(-2147483648))
    zero = jnp.zeros((BATCH, 1), jnp.int32)
    cur = jnp.where(count_ge(zero) >= PRE, zero, big)

    def bis_body(k, cur):
        bit = jnp.int32(1) << (jnp.int32(30) - k)
        cand = cur | bit
        return jnp.where(count_ge(cand) >= PRE, cand, cur)

    thr = jax.lax.fori_loop(0, 31, bis_body, cur)

    gt = jnp.sum((key > thr).astype(jnp.int32), axis=1, keepdims=True)
    need = PRE - gt
    eq = key == thr

    def idx_body(k, cur):
        bit = jnp.int32(1) << (jnp.int32(14) - k)
        cand = cur | bit
        cnt = jnp.sum((eq & (lane < cand)).astype(jnp.int32), axis=1,
                      keepdims=True)
        return jnp.where(cnt < need, cand, cur)

    idx_thr = jax.lax.fori_loop(0, 15, idx_body,
                                jnp.zeros((BATCH, 1), jnp.int32))
    valid = (key > thr) | (eq & (lane <= idx_thr))

    a_y1 = anchors_ref[0:1, :]
    a_x1 = anchors_ref[1:2, :]
    a_y2 = anchors_ref[2:3, :]
    a_x2 = anchors_ref[3:4, :]
    anc_w = a_x2 - a_x1
    anc_h = a_y2 - a_y1
    anc_cx = a_x1 + 0.5 * anc_w
    anc_cy = a_y1 + 0.5 * anc_h
    d_y = deltas_ref[0] * jnp.float32(0.1)
    d_x = deltas_ref[1] * jnp.float32(0.1)
    d_h = deltas_ref[2] * jnp.float32(0.2)
    d_w = deltas_ref[3] * jnp.float32(0.2)
    bb_w = jnp.exp(d_w) * anc_w
    bb_h = jnp.exp(d_h) * anc_h
    bb_cx = d_x * anc_w + anc_cx
    bb_cy = d_y * anc_h + anc_cy
    y1 = bb_cy - 0.5 * bb_h
    x1 = bb_cx - 0.5 * bb_w
    y2 = bb_h + y1
    x2 = bb_w + x1

    zf = jnp.float32(0.0)
    pad = jnp.zeros((BATCH, PADN - N), jnp.float32)
    vf = jnp.concatenate([jnp.where(valid, jnp.float32(1.0), zf), pad],
                         axis=1)
    V = vf.reshape(BATCH * BLKS, 128)
    li = jax.lax.broadcasted_iota(jnp.int32, (128, 128), 0)
    lj = jax.lax.broadcasted_iota(jnp.int32, (128, 128), 1)
    T128 = jnp.where(li < lj, jnp.float32(1.0), zf)
    rank_ex = jax.lax.dot_general(
        V, T128, (((1,), (0,)), ((), ())),
        preferred_element_type=jnp.float32,
        precision=jax.lax.Precision.HIGHEST)
    kidx_f = jax.lax.broadcasted_iota(
        jnp.int32, (BATCH * BLKS, 128), 1).astype(jnp.float32)
    lo = jnp.zeros((BATCH * BLKS, 128), jnp.int32)
    for bit in (64, 32, 16, 8, 4, 2, 1):
        c = lo + bit
        rc = jnp.take_along_axis(rank_ex, c, axis=1)
        lo = jnp.where(rc <= kidx_f, c, lo)
    cnt = rank_ex[:, 127:128] + V[:, 127:128]
    keep = kidx_f < cnt

    C = jnp.sum(vf.reshape(BATCH, BLKS, 128), axis=2)
    i160 = jax.lax.broadcasted_iota(jnp.int32, (BLKS, BLKS), 0)
    j160 = jax.lax.broadcasted_iota(jnp.int32, (BLKS, BLKS), 1)
    T160 = jnp.where(i160 < j160, jnp.float32(1.0), zf)
    O = jax.lax.dot_general(
        C, T160, (((1,), (0,)), ((), ())),
        preferred_element_type=jnp.float32,
        precision=jax.lax.Precision.HIGHEST)
    Oi = O.astype(jnp.int32)
    shift = jnp.broadcast_to((Oi & 127)[:, :, None],
                             (BATCH, BLKS, 128)).reshape(BATCH * BLKS, 128)
    lane128 = jax.lax.broadcasted_iota(jnp.int32, (BATCH * BLKS, 128), 1)
    idxrot = (lane128 - shift + 128) & 127
    real = idxrot.astype(jnp.float32) < cnt
    partA = real & (lane128 >= shift)
    partB = real & (lane128 < shift)
    m0 = Oi >> 7
    mm = jax.lax.broadcasted_iota(jnp.int32, (BATCH, BLKS, OROWS), 2)
    RA = jnp.where(m0[:, :, None] == mm, jnp.float32(1.0), zf)
    RB = jnp.where((m0 + 1)[:, :, None] == mm, jnp.float32(1.0), zf)
    glo = jnp.take_along_axis(lo, idxrot, axis=1)

    def compact(p):
        pp = jnp.concatenate([p, pad], axis=1).reshape(BATCH * BLKS, 128)
        rot = jnp.take_along_axis(pp, glo, axis=1)
        mA = jnp.where(partA, rot, zf).reshape(BATCH, BLKS, 128)
        mB = jnp.where(partB, rot, zf).reshape(BATCH, BLKS, 128)
        o3 = (jax.lax.dot_general(
                  RA, mA, (((1,), (1,)), ((0,), (0,))),
                  preferred_element_type=jnp.float32,
                  precision=jax.lax.Precision.HIGHEST)
              + jax.lax.dot_general(
                  RB, mB, (((1,), (1,)), ((0,), (0,))),
                  preferred_element_type=jnp.float32,
                  precision=jax.lax.Precision.HIGHEST))
        return o3.reshape(BATCH, CW)

    clane = jax.lax.broadcasted_iota(jnp.int32, (BATCH, CW), 1)
    yy1 = compact(y1)
    xx1 = compact(x1)
    yy2 = compact(y2)
    xx2 = compact(x2)
    s_c = jnp.where(clane < PRE, compact(scores), jnp.float32(-1.0))
    ab = (jnp.float32(IOU_THR)
          * jnp.maximum(yy2 - yy1, 0.0) * jnp.maximum(xx2 - xx1, 0.0))

    s_ref[...] = s_c
    out_ref[...] = jnp.zeros((4, BATCH, OUT_PAD), jnp.float32)
    out_lane = jax.lax.broadcasted_iota(jnp.int32, (BATCH, OUT_PAD), 1)
    lane = clane

    def one_pick(s):
        m = jnp.max(s, axis=1, keepdims=True)
        anyv = m >= 0.0
        pick = (s == m) & anyv
        pos = jnp.min(jnp.where(pick, lane, jnp.int32(CW)), axis=1,
                      keepdims=True)
        onehot = lane == pos
        oh_f = jnp.where(onehot, jnp.float32(1.0), jnp.float32(0.0))

        def sel(plane):
            return jnp.sum(plane * oh_f, axis=1, keepdims=True)

        by1 = sel(yy1)
        bx1 = sel(xx1)
        by2 = sel(yy2)
        bx2 = sel(xx2)

        inter = (jnp.maximum(jnp.minimum(by2, yy2) - jnp.maximum(by1, yy1),
                             0.0)
                 * jnp.maximum(jnp.minimum(bx2, xx2) - jnp.maximum(bx1, xx1),
                               0.0))
        area_a = jnp.maximum(by2 - by1, 0.0) * jnp.maximum(bx2 - bx1, 0.0)
        supp = (jnp.float32(1.0 + IOU_THR) * inter
                > jnp.float32(IOU_THR) * area_a + ab)
        supp = supp | onehot
        s_next = jnp.where(anyv & supp, jnp.float32(-1.0), s)
        return s_next, (by1, bx1, by2, bx2), anyv

    def write_out(o, i, box, anyv):
        wmask = (out_lane == i) & anyv
        return [jnp.where(wmask, jnp.clip(b, 0.0, 1.0), oo)
                for b, oo in zip(box, o)]

    def body(i, _):
        s = s_ref[...]
        o = [out_ref[0], out_ref[1], out_ref[2], out_ref[3]]
        for k in range(6):
            s, box, anyv = one_pick(s)
            o = write_out(o, 6 * i + k, box, anyv)
        s_ref[...] = s
        out_ref[...] = jnp.stack(o, axis=0)
        return 0

    jax.lax.fori_loop(0, POST // 6, body, 0)


@jax.jit
def kernel(rpn_bbox_deltas, rpn_labels, anchors):
    deltas_t = jnp.transpose(rpn_bbox_deltas, (2, 0, 1))
    anchors_t = jnp.transpose(anchors, (1, 0))
    out = pl.pallas_call(
        _nms_kernel,
        out_shape=jax.ShapeDtypeStruct((4, BATCH, OUT_PAD), jnp.float32),
        scratch_shapes=[pltpu.VMEM((BATCH, CW), jnp.float32)],
    )(rpn_labels, deltas_t, anchors_t)
    return jnp.transpose(out[:, :, :POST], (1, 2, 0))
